# Initial kernel scaffold; baseline (speedup 1.0000x reference)
import jax
import jax.numpy as jnp
from jax.experimental import pallas as pl


def kernel(adj1, adj2, W1, W2, W3, W4, W5, W6, L1w, L1b, L2w, L2b, L3w, L3b):
    raise NotImplementedError("write your pallas kernel here")


# speedup vs baseline: 1.4294x; 1.4294x over previous
import functools

import jax
import jax.numpy as jnp
from jax.experimental import pallas as pl
from jax.experimental.pallas import tpu as pltpu

N = 4096
NH = 128
NL = 6
BR = 256
NB = N // BR


def _mlp_block(t, L1w_ref, L1b_ref, L2w_ref, L2b_ref, L3wT_ref, L3b_ref):
    h1 = jnp.dot(t.astype(jnp.bfloat16), L1w_ref[...],
                 preferred_element_type=jnp.float32) + L1b_ref[...]
    h1 = jnp.maximum(h1, 0.0)
    h2 = jnp.dot(h1.astype(jnp.bfloat16), L2w_ref[...],
                 preferred_element_type=jnp.float32) + L2b_ref[...]
    h2 = jnp.maximum(h2, 0.0)
    return (jnp.sum(h2 * L3wT_ref[...], axis=1, keepdims=True)
            + L3b_ref[0, 0])


def _gnn_kernel(adj_ref, s_other_ref, W1_ref, Wstack_ref,
                L1w_ref, L1b_ref, L2w_ref, L2b_ref, L3wT_ref, L3b_ref,
                out_ref,
                adj_sc, y_sc, c_sc, r_sc):
    l = pl.program_id(0)
    i = pl.program_id(1)
    rows = pl.ds(i * BR, BR)

    @pl.when(l == 0)
    def _():
        adj_sc[rows, :] = adj_ref[...].astype(jnp.bfloat16)

    @pl.when((l == 0) & (i == 0))
    def _():
        y_sc[...] = W1_ref[...]

    @pl.when((l >= 1) & (i == 0))
    def _():
        W = Wstack_ref[l - 1]
        y_sc[...] = jnp.dot(c_sc[...].astype(jnp.bfloat16), W,
                            preferred_element_type=jnp.float32
                            ).astype(jnp.bfloat16)

    z = jnp.dot(adj_sc[rows, :], y_sc[...],
                preferred_element_type=jnp.float32)
    act = jnp.maximum(z, 0.0)

    nrm = jnp.sqrt(jnp.sum(act * act, axis=1, keepdims=True))
    n = jnp.where(l < NL - 1, act / jnp.maximum(nrm, 1e-12), act)

    c_sc[rows, :] = n
    r_new = jnp.where(l == 0, 0.0, r_sc[rows, :]) + n
    r_sc[rows, :] = r_new

    mlp = functools.partial(_mlp_block, L1w_ref=L1w_ref, L1b_ref=L1b_ref,
                            L2w_ref=L2w_ref, L2b_ref=L2b_ref,
                            L3wT_ref=L3wT_ref, L3b_ref=L3b_ref)
    s_new = jnp.where(l == 0, 0.0, out_ref[rows, :]) + mlp(n)
    out_ref[rows, :] = s_new

    @pl.when(l == NL - 1)
    def _():
        s_fin = s_new + mlp(r_new)
        out_ref[rows, :] = s_fin * (1.0 / 7.0) * s_other_ref[rows, :]


def _gnn_graph(adj, s_other, W1c, Wstack, L1wc, L1b2, L2wc, L2b2, L3wT, L3b2):
    return pl.pallas_call(
        _gnn_kernel,
        grid=(NL, NB),
        in_specs=[
            pl.BlockSpec((BR, N), lambda l, i: (jnp.where(l == 0, i, 0), 0)),
            pl.BlockSpec((N, 1), lambda l, i: (0, 0)),
            pl.BlockSpec((N, NH), lambda l, i: (0, 0)),
            pl.BlockSpec((NL - 1, NH, NH), lambda l, i: (0, 0, 0)),
            pl.BlockSpec((NH, 2 * NH), lambda l, i: (0, 0)),
            pl.BlockSpec((1, 2 * NH), lambda l, i: (0, 0)),
            pl.BlockSpec((2 * NH, 2 * NH), lambda l, i: (0, 0)),
            pl.BlockSpec((1, 2 * NH), lambda l, i: (0, 0)),
            pl.BlockSpec((1, 2 * NH), lambda l, i: (0, 0)),
            pl.BlockSpec((1, 1), lambda l, i: (0, 0)),
        ],
        out_specs=pl.BlockSpec((N, 1), lambda l, i: (0, 0)),
        out_shape=jax.ShapeDtypeStruct((N, 1), jnp.float32),
        scratch_shapes=[
            pltpu.VMEM((N, N), jnp.bfloat16),
            pltpu.VMEM((N, NH), jnp.bfloat16),
            pltpu.VMEM((N, NH), jnp.float32),
            pltpu.VMEM((N, NH), jnp.float32),
        ],
        compiler_params=pltpu.CompilerParams(
            dimension_semantics=("arbitrary", "arbitrary"),
            vmem_limit_bytes=62 * 1024 * 1024,
        ),
    )(adj, s_other, W1c, Wstack, L1wc, L1b2, L2wc, L2b2, L3wT, L3b2)


def kernel(adj1, adj2, W1, W2, W3, W4, W5, W6, L1w, L1b, L2w, L2b, L3w, L3b):
    bf = jnp.bfloat16
    W1c = W1.astype(bf)
    Wstack = jnp.stack([W2, W3, W4, W5, W6]).astype(bf)
    L1wc = L1w.astype(bf)
    L2wc = L2w.astype(bf)
    L1b2 = L1b.reshape(1, -1)
    L2b2 = L2b.reshape(1, -1)
    L3wT = L3w.reshape(1, -1)
    L3b2 = L3b.reshape(1, 1)
    ones = jnp.ones((N, 1), jnp.float32)
    s1 = _gnn_graph(adj1, ones, W1c, Wstack, L1wc, L1b2, L2wc, L2b2, L3wT, L3b2)
    return _gnn_graph(adj2, s1, W1c, Wstack, L1wc, L1b2, L2wc, L2b2, L3wT, L3b2)
